# baseline (speedup 1.0000x reference)
import functools
import math

import jax
import jax.numpy as jnp
from jax import lax
from jax.experimental import pallas as pl
from jax.experimental.pallas import tpu as pltpu
from jax.experimental.pallas import tpu_sc as plsc

jax.config.update("jax_enable_x64", True)

_EMBED = 256
_EP = _EMBED // 2
_CB = 512
_HMASK = -65536
_P1, _P2, _P3 = 73856093, 19349663, 83492791

_GRID_DIM = max(2, int(math.pow(_CB, 1 / 3)))
_SCALES = []
_curr = 50.0
for _ in range(4):
    _SCALES.append(_curr)
    _curr /= _GRID_DIM

_L = 16
_B = 96
_NW = 32


def _floor_i32(t):
    i = t.astype(jnp.int32)
    return jnp.where(i.astype(jnp.float32) > t, i - 1, i)


def _mod_pos(x, s):
    r = lax.rem(x, jnp.float32(s))
    return jnp.where(r < 0, r + jnp.float32(s), r)


def _make_sc_quant(n2):
    rows_per_tile = n2 // _NW
    chunks = rows_per_tile // _B
    assert chunks % 2 == 0
    info = plsc.get_sparse_core_info()
    nc = info.num_cores
    mesh = plsc.VectorSubcoreMesh(core_axis_name="c", subcore_axis_name="s")

    @functools.partial(
        pl.kernel,
        mesh=mesh,
        out_type=[
            jax.ShapeDtypeStruct((n2, _EP), jnp.int32),
            jax.ShapeDtypeStruct((n2,), jnp.int32),
            jax.ShapeDtypeStruct((n2,), jnp.int32),
        ],
        scratch_types=[
            pltpu.VMEM((rows_per_tile,), jnp.float32),
            pltpu.VMEM((rows_per_tile,), jnp.float32),
            pltpu.VMEM((rows_per_tile,), jnp.float32),
        ] + [pltpu.VMEM((_B,), jnp.int32)] * 12
          + [pltpu.VMEM((_B, _EP), jnp.int32)] * 8
          + [pltpu.SemaphoreType.DMA] * 4,
    )
    def sc_quant(cx_h, cy_h, cz_h, e0_h, e1_h, e2_h, e3_h,
                 temb_h, lo_h, hi_h,
                 cxt, cyt, czt,
                 i0a, i1a, i2a, i3a, loa, hia,
                 i0b, i1b, i2b, i3b, lob, hib,
                 g0a, g1a, g2a, g3a, g0b, g1b, g2b, g3b,
                 gsem0, gsem1, wsem0, wsem1):
        wid = lax.axis_index("s") * nc + lax.axis_index("c")
        tile_base = pl.multiple_of(wid * jnp.int32(rows_per_tile), _B)
        pltpu.sync_copy(cx_h.at[pl.ds(tile_base, rows_per_tile)], cxt)
        pltpu.sync_copy(cy_h.at[pl.ds(tile_base, rows_per_tile)], cyt)
        pltpu.sync_copy(cz_h.at[pl.ds(tile_base, rows_per_tile)], czt)

        slots = (
            ((i0a, i1a, i2a, i3a), (loa, hia), (g0a, g1a, g2a, g3a),
             gsem0, wsem0),
            ((i0b, i1b, i2b, i3b), (lob, hib), (g0b, g1b, g2b, g3b),
             gsem1, wsem1),
        )

        def hbase(c):
            return pl.multiple_of(tile_base + c * jnp.int32(_B), _B)

        def drain_wb(slot, c):
            _, (lo_v, hi_v), gb, _, wsem = slots[slot]
            pltpu.make_async_copy(gb[0], temb_h.at[pl.ds(hbase(c), _B)],
                                  wsem).wait()
            pltpu.make_async_copy(lo_v, lo_h.at[pl.ds(hbase(c), _B)],
                                  wsem).wait()
            pltpu.make_async_copy(hi_v, hi_h.at[pl.ds(hbase(c), _B)],
                                  wsem).wait()

        def compute_and_fire(slot, c):
            (i0, i1, i2, i3), (lo_v, hi_v), gb, gsem, _ = slots[slot]
            off = c * jnp.int32(_B)
            for i in range(_B // _L):
                sl = pl.ds(i * _L, _L)
                src = pl.ds(off + i * _L, _L)
                x, y, z = cxt[src], cyt[src], czt[src]
                fs = []
                for l in range(4):
                    s = _SCALES[l]
                    gx = _floor_i32(x / jnp.float32(s))
                    gy = _floor_i32(y / jnp.float32(s))
                    gz = _floor_i32(z / jnp.float32(s))
                    fl = (gx * _P1 + gy * _P2 + gz * _P3) & (_CB - 1)
                    fs.append(fl)
                    if l < 3:
                        x = _mod_pos(x, s)
                        y = _mod_pos(y, s)
                        z = _mod_pos(z, s)
                i0[sl], i1[sl], i2[sl], i3[sl] = fs
                lo_v[sl] = (((fs[0] & 31) << 27) | (fs[1] << 18)
                            | (fs[2] << 9) | fs[3])
                hi_v[sl] = fs[0] >> 5
            pltpu.async_copy(e0_h.at[i0], gb[0], gsem)
            pltpu.async_copy(e1_h.at[i1], gb[1], gsem)
            pltpu.async_copy(e2_h.at[i2], gb[2], gsem)
            pltpu.async_copy(e3_h.at[i3], gb[3], gsem)

        def process(slot, c):
            (i0, i1, i2, i3), (lo_v, hi_v), gb, gsem, wsem = slots[slot]
            pltpu.make_async_copy(e0_h.at[i0], gb[0], gsem).wait()
            pltpu.make_async_copy(e1_h.at[i1], gb[1], gsem).wait()
            pltpu.make_async_copy(e2_h.at[i2], gb[2], gsem).wait()
            pltpu.make_async_copy(e3_h.at[i3], gb[3], gsem).wait()

            def accum_row(r, carry):
                for k in range(_EP // _L):
                    sk = pl.ds(k * _L, _L)
                    vs = [gb[j][r, sk] for j in range(4)]
                    los = [lax.bitcast_convert_type(v << 16, jnp.float32)
                           for v in vs]
                    his = [lax.bitcast_convert_type(v & _HMASK, jnp.float32)
                           for v in vs]
                    lo_s = (los[0] + los[1]) + (los[2] + los[3])
                    hi_s = (his[0] + his[1]) + (his[2] + his[3])
                    lo_i = lax.bitcast_convert_type(lo_s, jnp.int32)
                    hi_i = lax.bitcast_convert_type(hi_s, jnp.int32)
                    gb[0][r, sk] = (hi_i & _HMASK) | (
                        lax.shift_right_logical(lo_i, jnp.int32(16)))
                return carry

            lax.fori_loop(jnp.int32(0), jnp.int32(_B), accum_row,
                          jnp.int32(0))
            base = hbase(c)
            pltpu.async_copy(gb[0], temb_h.at[pl.ds(base, _B)], wsem)
            pltpu.async_copy(lo_v, lo_h.at[pl.ds(base, _B)], wsem)
            pltpu.async_copy(hi_v, hi_h.at[pl.ds(base, _B)], wsem)

        def pair(g, carry):
            for b in range(2):
                c = g * jnp.int32(2) + jnp.int32(b)

                @pl.when(c >= 2)
                def _():
                    drain_wb(b, c)

                compute_and_fire(b, c)

                @pl.when(c >= 1)
                def _():
                    process(1 - b, c - 1)
            return carry

        lax.fori_loop(jnp.int32(0), jnp.int32(chunks // 2), pair,
                      jnp.int32(0))
        last = jnp.int32(chunks - 1)
        process(1, last)
        drain_wb(0, last)
        drain_wb(1, last)

    return sc_quant


def _fusion_body(x_ref, w_ref, b_ref, g_ref, bt_ref, o_ref):
    x = x_ref[...].astype(jnp.float32)
    h = lax.dot_general(x, w_ref[...], (((1,), (1,)), ((), ())),
                        preferred_element_type=jnp.float32)
    h = h + b_ref[...]
    m = jnp.mean(h, axis=-1, keepdims=True)
    v = jnp.mean((h - m) ** 2, axis=-1, keepdims=True)
    h = (h - m) / jnp.sqrt(v + 1e-5) * g_ref[...] + bt_ref[...]
    o_ref[...] = jnp.maximum(h, 0.0)


def kernel(coord, emb0, emb1, emb2, emb3, W, b, gamma, beta):
    n = coord.shape[0]
    chunk_rows = _NW * _B * 2
    n2 = ((n + chunk_rows - 1) // chunk_rows) * chunk_rows
    coordp = jnp.pad(coord.astype(jnp.float32), ((0, n2 - n), (0, 0)))
    cx, cy, cz = coordp[:, 0], coordp[:, 1], coordp[:, 2]
    ei32 = [
        lax.bitcast_convert_type(
            e.astype(jnp.bfloat16).reshape(_CB, _EP, 2), jnp.int32)
        for e in (emb0, emb1, emb2, emb3)
    ]
    temb_i, lo, hi = _make_sc_quant(n2)(cx, cy, cz, *ei32)
    temb = lax.bitcast_convert_type(temb_i, jnp.bfloat16).reshape(n2, _EMBED)

    bn = 800
    grid = n // bn
    out = pl.pallas_call(
        _fusion_body,
        grid=(grid,),
        in_specs=[
            pl.BlockSpec((bn, _EMBED), lambda i: (i, i - i)),
            pl.BlockSpec((_EMBED, _EMBED), lambda i: (i - i, i - i)),
            pl.BlockSpec((1, _EMBED), lambda i: (i - i, i - i)),
            pl.BlockSpec((1, _EMBED), lambda i: (i - i, i - i)),
            pl.BlockSpec((1, _EMBED), lambda i: (i - i, i - i)),
        ],
        out_specs=pl.BlockSpec((bn, _EMBED), lambda i: (i, i - i)),
        out_shape=jax.ShapeDtypeStruct((n, _EMBED), jnp.float32),
    )(temb, W, b[None, :], gamma[None, :], beta[None, :])

    lo64 = lo[:n].astype(jnp.int64) & 0xFFFFFFFF
    cid = (hi[:n].astype(jnp.int64) << 32) | lo64
    return (out, cid)


# speedup vs baseline: 1.3948x; 1.3948x over previous
import functools
import math

import jax
import jax.numpy as jnp
from jax import lax
from jax.experimental import pallas as pl
from jax.experimental.pallas import tpu as pltpu
from jax.experimental.pallas import tpu_sc as plsc

jax.config.update("jax_enable_x64", True)

_EMBED = 256
_CB = 512
_P1, _P2, _P3 = 73856093, 19349663, 83492791

_GRID_DIM = max(2, int(math.pow(_CB, 1 / 3)))
_SCALES = []
_curr = 50.0
for _ in range(4):
    _SCALES.append(_curr)
    _curr /= _GRID_DIM

_L = 16
_B = 48
_NW = 32


def _floor_i32(t):
    i = t.astype(jnp.int32)
    return jnp.where(i.astype(jnp.float32) > t, i - 1, i)


def _mod_pos(x, s):
    r = lax.rem(x, jnp.float32(s))
    return jnp.where(r < 0, r + jnp.float32(s), r)


def _make_sc_quant(n2):
    rows_per_tile = n2 // _NW
    chunks = rows_per_tile // _B
    assert chunks % 2 == 0
    info = plsc.get_sparse_core_info()
    nc = info.num_cores
    mesh = plsc.VectorSubcoreMesh(core_axis_name="c", subcore_axis_name="s")

    @functools.partial(
        pl.kernel,
        mesh=mesh,
        out_type=[
            jax.ShapeDtypeStruct((n2, _EMBED), jnp.float32),
            jax.ShapeDtypeStruct((n2,), jnp.int32),
            jax.ShapeDtypeStruct((n2,), jnp.int32),
        ],
        scratch_types=[
            pltpu.VMEM((rows_per_tile,), jnp.float32),
            pltpu.VMEM((rows_per_tile,), jnp.float32),
            pltpu.VMEM((rows_per_tile,), jnp.float32),
        ] + [pltpu.VMEM((_B,), jnp.int32)] * 12
          + [pltpu.VMEM((_B, _EMBED), jnp.float32)] * 8
          + [pltpu.SemaphoreType.DMA] * 4,
    )
    def sc_quant(cx_h, cy_h, cz_h, e0_h, e1_h, e2_h, e3_h,
                 temb_h, lo_h, hi_h,
                 cxt, cyt, czt,
                 i0a, i1a, i2a, i3a, loa, hia,
                 i0b, i1b, i2b, i3b, lob, hib,
                 g0a, g1a, g2a, g3a, g0b, g1b, g2b, g3b,
                 gsem0, gsem1, wsem0, wsem1):
        wid = lax.axis_index("s") * nc + lax.axis_index("c")
        tile_base = pl.multiple_of(wid * jnp.int32(rows_per_tile), _B)
        pltpu.sync_copy(cx_h.at[pl.ds(tile_base, rows_per_tile)], cxt)
        pltpu.sync_copy(cy_h.at[pl.ds(tile_base, rows_per_tile)], cyt)
        pltpu.sync_copy(cz_h.at[pl.ds(tile_base, rows_per_tile)], czt)

        slots = (
            ((i0a, i1a, i2a, i3a), (loa, hia), (g0a, g1a, g2a, g3a),
             gsem0, wsem0),
            ((i0b, i1b, i2b, i3b), (lob, hib), (g0b, g1b, g2b, g3b),
             gsem1, wsem1),
        )

        def hbase(c):
            return pl.multiple_of(tile_base + c * jnp.int32(_B), _B)

        def drain_wb(slot, c):
            _, (lo_v, hi_v), gb, _, wsem = slots[slot]
            pltpu.make_async_copy(gb[0], temb_h.at[pl.ds(hbase(c), _B)],
                                  wsem).wait()
            pltpu.make_async_copy(lo_v, lo_h.at[pl.ds(hbase(c), _B)],
                                  wsem).wait()
            pltpu.make_async_copy(hi_v, hi_h.at[pl.ds(hbase(c), _B)],
                                  wsem).wait()

        def compute_and_fire(slot, c):
            (i0, i1, i2, i3), (lo_v, hi_v), gb, gsem, _ = slots[slot]
            off = c * jnp.int32(_B)
            for i in range(_B // _L):
                sl = pl.ds(i * _L, _L)
                src = pl.ds(off + i * _L, _L)
                x, y, z = cxt[src], cyt[src], czt[src]
                fs = []
                for l in range(4):
                    s = _SCALES[l]
                    gx = _floor_i32(x / jnp.float32(s))
                    gy = _floor_i32(y / jnp.float32(s))
                    gz = _floor_i32(z / jnp.float32(s))
                    fl = (gx * _P1 + gy * _P2 + gz * _P3) & (_CB - 1)
                    fs.append(fl)
                    if l < 3:
                        x = _mod_pos(x, s)
                        y = _mod_pos(y, s)
                        z = _mod_pos(z, s)
                i0[sl], i1[sl], i2[sl], i3[sl] = fs
                lo_v[sl] = (((fs[0] & 31) << 27) | (fs[1] << 18)
                            | (fs[2] << 9) | fs[3])
                hi_v[sl] = fs[0] >> 5
            hh = _B // 2
            for eh, iv, gbj in ((e0_h, i0, gb[0]), (e1_h, i1, gb[1]),
                                (e2_h, i2, gb[2]), (e3_h, i3, gb[3])):
                pltpu.async_copy(eh.at[iv.at[pl.ds(0, hh)]],
                                 gbj.at[pl.ds(0, hh)], gsem)
                pltpu.async_copy(eh.at[iv.at[pl.ds(hh, hh)]],
                                 gbj.at[pl.ds(hh, hh)], gsem)

        def process(slot, c):
            (i0, i1, i2, i3), (lo_v, hi_v), gb, gsem, wsem = slots[slot]
            hh = _B // 2
            for eh, iv, gbj in ((e0_h, i0, gb[0]), (e1_h, i1, gb[1]),
                                (e2_h, i2, gb[2]), (e3_h, i3, gb[3])):
                pltpu.make_async_copy(eh.at[iv.at[pl.ds(0, hh)]],
                                      gbj.at[pl.ds(0, hh)], gsem).wait()
                pltpu.make_async_copy(eh.at[iv.at[pl.ds(hh, hh)]],
                                      gbj.at[pl.ds(hh, hh)], gsem).wait()

            def accum_row(r, carry):
                for k in range(_EMBED // _L):
                    sk = pl.ds(k * _L, _L)
                    vs = [gb[j][r, sk] for j in range(4)]
                    acc = (vs[0] + vs[1]) + (vs[2] + vs[3])
                    gb[0][r, sk] = acc
                return carry

            lax.fori_loop(jnp.int32(0), jnp.int32(_B), accum_row,
                          jnp.int32(0))
            base = hbase(c)
            pltpu.async_copy(gb[0], temb_h.at[pl.ds(base, _B)], wsem)
            pltpu.async_copy(lo_v, lo_h.at[pl.ds(base, _B)], wsem)
            pltpu.async_copy(hi_v, hi_h.at[pl.ds(base, _B)], wsem)

        def pair(g, carry):
            for b in range(2):
                c = g * jnp.int32(2) + jnp.int32(b)

                @pl.when(c >= 2)
                def _():
                    drain_wb(b, c)

                compute_and_fire(b, c)

                @pl.when(c >= 1)
                def _():
                    process(1 - b, c - 1)
            return carry

        lax.fori_loop(jnp.int32(0), jnp.int32(chunks // 2), pair,
                      jnp.int32(0))
        last = jnp.int32(chunks - 1)
        process(1, last)
        drain_wb(0, last)
        drain_wb(1, last)

    return sc_quant


def _fusion_body(x_ref, w_ref, b_ref, g_ref, bt_ref, o_ref):
    x = x_ref[...].astype(jnp.float32)
    h = lax.dot_general(x, w_ref[...], (((1,), (1,)), ((), ())),
                        preferred_element_type=jnp.float32)
    h = h + b_ref[...]
    m = jnp.mean(h, axis=-1, keepdims=True)
    v = jnp.mean((h - m) ** 2, axis=-1, keepdims=True)
    h = (h - m) / jnp.sqrt(v + 1e-5) * g_ref[...] + bt_ref[...]
    o_ref[...] = jnp.maximum(h, 0.0)


def kernel(coord, emb0, emb1, emb2, emb3, W, b, gamma, beta):
    n = coord.shape[0]
    chunk_rows = _NW * _B * 2
    n2 = ((n + chunk_rows - 1) // chunk_rows) * chunk_rows
    coordp = jnp.pad(coord.astype(jnp.float32), ((0, n2 - n), (0, 0)))
    cx, cy, cz = coordp[:, 0], coordp[:, 1], coordp[:, 2]
    temb, lo, hi = _make_sc_quant(n2)(cx, cy, cz, emb0, emb1, emb2, emb3)

    bn = 800
    grid = n // bn
    out = pl.pallas_call(
        _fusion_body,
        grid=(grid,),
        in_specs=[
            pl.BlockSpec((bn, _EMBED), lambda i: (i, i - i)),
            pl.BlockSpec((_EMBED, _EMBED), lambda i: (i - i, i - i)),
            pl.BlockSpec((1, _EMBED), lambda i: (i - i, i - i)),
            pl.BlockSpec((1, _EMBED), lambda i: (i - i, i - i)),
            pl.BlockSpec((1, _EMBED), lambda i: (i - i, i - i)),
        ],
        out_specs=pl.BlockSpec((bn, _EMBED), lambda i: (i, i - i)),
        out_shape=jax.ShapeDtypeStruct((n, _EMBED), jnp.float32),
    )(temb, W, b[None, :], gamma[None, :], beta[None, :])

    lo64 = lo[:n].astype(jnp.int64) & 0xFFFFFFFF
    cid = (hi[:n].astype(jnp.int64) << 32) | lo64
    return (out, cid)
